# baseline (speedup 1.0000x reference)
import jax
import jax.numpy as jnp
from jax import lax
from jax.experimental import pallas as pl
from jax.experimental.pallas import tpu as pltpu
from jax.experimental.pallas import tpu_sc as plsc

S = 4096
E = S // 2
F = 256
NC = 2
NS = 16
NW = NC * NS
L = 16
CHUNK = S // NW
STRIPE = 2 * F // NS
BLK = 2048


def _proj_body(x_ref, w_ref, o_ref):
    pm = jnp.dot(x_ref[...], w_ref[...], preferred_element_type=jnp.float32)
    o_ref[0] = pm[:, 0:128]
    o_ref[1] = pm[:, 128:256]


_tc_proj = pl.pallas_call(
    _proj_body,
    grid=(S // BLK,),
    in_specs=[
        pl.BlockSpec((BLK, F), lambda i: (i, 0)),
        pl.BlockSpec((F, F), lambda i: (0, 0)),
    ],
    out_specs=pl.BlockSpec((2, BLK, 128), lambda i: (0, i, 0)),
    out_shape=jax.ShapeDtypeStruct((2, S, 128), jnp.float32),
)


def _sc_body(h_hbm, edges_hbm, out_p, out_cnt,
             e_v, recv_v, h0_v, h1_v, z_v, cnt_v, acc_sh,
             sem_in, sem_x0, sem_x1):
    c = lax.axis_index("c")
    sid = lax.axis_index("s")
    w = c * NS + sid
    e0 = lax.rem(w, NS) * CHUNK
    cp_e = pltpu.async_copy(edges_hbm.at[pl.ds(e0, CHUNK), :], e_v, sem_in)
    cp_x0 = pltpu.async_copy(
        h_hbm.at[0, pl.ds(w * CHUNK, CHUNK), :], h0_v, sem_x0)
    cp_x1 = pltpu.async_copy(
        h_hbm.at[1, pl.ds(w * CHUNK, CHUNK), :], h1_v, sem_x1)
    zero = jnp.zeros((L,), jnp.float32)

    def _zero(i, carry):
        for k in range(128 // L):
            z_v[i, pl.ds(k * L, L)] = zero
        return carry
    lax.fori_loop(0, STRIPE, _zero, 0)
    for r in (0, 8):
        for k in range(128 // L):
            cnt_v[r, pl.ds(k * L, L)] = zero
    cp_e.wait()
    col = jnp.where(w < NS, jnp.int32(1), jnp.int32(0))
    cols = jnp.zeros((L,), jnp.int32) + col
    for i in range(CHUNK // L):
        rows = lax.iota(jnp.int32, L) + jnp.int32(i * L)
        g = plsc.load_gather(e_v, [rows, cols])
        recv_v[0, pl.ds(i * L, L)] = g
        recv_v[1, pl.ds(i * L, L)] = g + jnp.int32(F)
    pltpu.sync_copy(z_v, acc_sh.at[pl.ds(sid * STRIPE, STRIPE), :])
    plsc.subcore_barrier()
    cp_x0.wait()
    pltpu.sync_copy(h0_v, acc_sh.at[recv_v.at[0]], add=True)
    cp_x1.wait()
    pltpu.sync_copy(h1_v, acc_sh.at[recv_v.at[1]], add=True)
    ones = jnp.ones((L,), jnp.float32)
    for i in range(CHUNK // L):
        f = recv_v[0, pl.ds(i * L, L)]
        plsc.addupdate_scatter(
            cnt_v, [(f // 128) * 8, lax.rem(f, 128)], ones)
    pltpu.sync_copy(cnt_v, out_cnt.at[pl.ds(w * 16, 16), :])
    plsc.subcore_barrier()
    pltpu.sync_copy(acc_sh.at[pl.ds(sid * STRIPE, STRIPE), :],
                    out_p.at[c, pl.ds(sid * STRIPE, STRIPE), :])


_sc_scatter = pl.kernel(
    _sc_body,
    out_type=[
        jax.ShapeDtypeStruct((NC, 2 * F, 128), jnp.float32),
        jax.ShapeDtypeStruct((NW * 16, 128), jnp.float32),
    ],
    mesh=plsc.VectorSubcoreMesh(core_axis_name="c", subcore_axis_name="s"),
    compiler_params=pltpu.CompilerParams(
        use_tc_tiling_on_sc=False, needs_layout_passes=False
    ),
    scratch_types=[
        pltpu.VMEM((CHUNK, 2), jnp.int32),
        pltpu.VMEM((2, CHUNK), jnp.int32),
        pltpu.VMEM((CHUNK, 128), jnp.float32),
        pltpu.VMEM((CHUNK, 128), jnp.float32),
        pltpu.VMEM((STRIPE, 128), jnp.float32),
        pltpu.VMEM((16, 128), jnp.float32),
        pltpu.VMEM_SHARED((2 * F, 128), jnp.float32),
        pltpu.SemaphoreType.DMA,
        pltpu.SemaphoreType.DMA,
        pltpu.SemaphoreType.DMA,
    ],
)


def _tc_body(p_ref, cnt_ref, dk_ref, b_ref, o_ref):
    pm = jnp.concatenate(
        [p_ref[0, 0:F] + p_ref[1, 0:F], p_ref[0, F:2 * F] + p_ref[1, F:2 * F]],
        axis=1)
    r = jnp.maximum(pm, 0.0)
    s = jnp.sum(jnp.reshape(cnt_ref[...], (NW, 16, 128)), axis=0)
    cnt = jnp.concatenate([s[0:1, :], s[8:9, :]], axis=1)
    tot = jnp.sum(r * cnt * dk_ref[0:F, :])
    o_ref[...] = jnp.reshape(tot + b_ref[0], (1, 1))


_tc_finish = pl.pallas_call(
    _tc_body,
    out_shape=jax.ShapeDtypeStruct((1, 1), jnp.float32),
    in_specs=[
        pl.BlockSpec(memory_space=pltpu.MemorySpace.VMEM),
        pl.BlockSpec(memory_space=pltpu.MemorySpace.VMEM),
        pl.BlockSpec(memory_space=pltpu.MemorySpace.VMEM),
        pl.BlockSpec(memory_space=pltpu.MemorySpace.SMEM),
    ],
)


def kernel(x, edges, W, dense_kernel, dense_bias):
    h2 = _tc_proj(x.astype(jnp.float32), W)
    parts, cnts = _sc_scatter(h2, edges)
    out = _tc_finish(parts, cnts, dense_kernel, dense_bias)
    return jnp.reshape(out, ())


# speedup vs baseline: 1.0057x; 1.0057x over previous
import jax
import jax.numpy as jnp
from jax import lax
from jax.experimental import pallas as pl
from jax.experimental.pallas import tpu as pltpu
from jax.experimental.pallas import tpu_sc as plsc

S = 4096
E = S // 2
F = 256
NC = 2
NS = 16
NW = NC * NS
L = 16
CHUNK = S // NW
STRIPE = 2 * F // NS
BLK = 2048


def _proj_body(x_ref, w_ref, o_ref):
    pm = jnp.dot(x_ref[...], w_ref[...], preferred_element_type=jnp.float32)
    o_ref[0] = pm[:, 0:128]
    o_ref[1] = pm[:, 128:256]


_tc_proj = pl.pallas_call(
    _proj_body,
    grid=(S // BLK,),
    in_specs=[
        pl.BlockSpec((BLK, F), lambda i: (i, 0)),
        pl.BlockSpec((F, F), lambda i: (0, 0)),
    ],
    out_specs=pl.BlockSpec((2, BLK, 128), lambda i: (0, i, 0)),
    out_shape=jax.ShapeDtypeStruct((2, S, 128), jnp.float32),
)


def _sc_body(h_hbm, edges_hbm, out_p, out_cnt,
             e_v, recv_v, h0_v, h1_v, z_v, cnt_v, acc_sh,
             sem_in, sem_x0, sem_x1):
    c = lax.axis_index("c")
    sid = lax.axis_index("s")
    w = c * NS + sid
    e0 = lax.rem(w, NS) * (2 * CHUNK)
    cp_e = pltpu.async_copy(edges_hbm.at[pl.ds(e0, 2 * CHUNK)], e_v, sem_in)
    cp_x0 = pltpu.async_copy(
        h_hbm.at[0, pl.ds(w * CHUNK, CHUNK), :], h0_v, sem_x0)
    cp_x1 = pltpu.async_copy(
        h_hbm.at[1, pl.ds(w * CHUNK, CHUNK), :], h1_v, sem_x1)
    zero = jnp.zeros((L,), jnp.float32)

    def _zero(i, carry):
        for k in range(128 // L):
            z_v[i, pl.ds(k * L, L)] = zero
        return carry
    lax.fori_loop(0, STRIPE, _zero, 0)
    for r in (0, 8):
        for k in range(128 // L):
            cnt_v[r, pl.ds(k * L, L)] = zero
    cp_e.wait()
    col = jnp.where(w < NS, jnp.int32(1), jnp.int32(0))
    cols = jnp.zeros((L,), jnp.int32) + col
    for i in range(CHUNK // L):
        rows = lax.iota(jnp.int32, L) + jnp.int32(i * L)
        g = plsc.load_gather(e_v, [rows * 2 + cols])
        recv_v[0, pl.ds(i * L, L)] = g
        recv_v[1, pl.ds(i * L, L)] = g + jnp.int32(F)
    pltpu.sync_copy(z_v, acc_sh.at[pl.ds(sid * STRIPE, STRIPE), :])
    plsc.subcore_barrier()
    cp_x0.wait()
    pltpu.sync_copy(h0_v, acc_sh.at[recv_v.at[0]], add=True)
    cp_x1.wait()
    pltpu.sync_copy(h1_v, acc_sh.at[recv_v.at[1]], add=True)
    ones = jnp.ones((L,), jnp.float32)
    for i in range(CHUNK // L):
        f = recv_v[0, pl.ds(i * L, L)]
        plsc.addupdate_scatter(
            cnt_v, [(f // 128) * 8, lax.rem(f, 128)], ones)
    pltpu.sync_copy(cnt_v, out_cnt.at[pl.ds(w * 16, 16), :])
    plsc.subcore_barrier()
    pltpu.sync_copy(acc_sh.at[pl.ds(sid * STRIPE, STRIPE), :],
                    out_p.at[c, pl.ds(sid * STRIPE, STRIPE), :])


_sc_scatter = pl.kernel(
    _sc_body,
    out_type=[
        jax.ShapeDtypeStruct((NC, 2 * F, 128), jnp.float32),
        jax.ShapeDtypeStruct((NW * 16, 128), jnp.float32),
    ],
    mesh=plsc.VectorSubcoreMesh(core_axis_name="c", subcore_axis_name="s"),
    compiler_params=pltpu.CompilerParams(
        use_tc_tiling_on_sc=False, needs_layout_passes=False
    ),
    scratch_types=[
        pltpu.VMEM((2 * CHUNK,), jnp.int32),
        pltpu.VMEM((2, CHUNK), jnp.int32),
        pltpu.VMEM((CHUNK, 128), jnp.float32),
        pltpu.VMEM((CHUNK, 128), jnp.float32),
        pltpu.VMEM((STRIPE, 128), jnp.float32),
        pltpu.VMEM((16, 128), jnp.float32),
        pltpu.VMEM_SHARED((2 * F, 128), jnp.float32),
        pltpu.SemaphoreType.DMA,
        pltpu.SemaphoreType.DMA,
        pltpu.SemaphoreType.DMA,
    ],
)


def _tc_body(p_ref, cnt_ref, dk_ref, b_ref, o_ref):
    pm = jnp.concatenate(
        [p_ref[0, 0:F] + p_ref[1, 0:F], p_ref[0, F:2 * F] + p_ref[1, F:2 * F]],
        axis=1)
    r = jnp.maximum(pm, 0.0)
    s = jnp.sum(jnp.reshape(cnt_ref[...], (NW, 16, 128)), axis=0)
    cnt = jnp.concatenate([s[0:1, :], s[8:9, :]], axis=1)
    tot = jnp.sum(r * cnt * dk_ref[0:F, :])
    o_ref[...] = jnp.reshape(tot + b_ref[0], (1, 1))


_tc_finish = pl.pallas_call(
    _tc_body,
    out_shape=jax.ShapeDtypeStruct((1, 1), jnp.float32),
    in_specs=[
        pl.BlockSpec(memory_space=pltpu.MemorySpace.VMEM),
        pl.BlockSpec(memory_space=pltpu.MemorySpace.VMEM),
        pl.BlockSpec(memory_space=pltpu.MemorySpace.VMEM),
        pl.BlockSpec(memory_space=pltpu.MemorySpace.SMEM),
    ],
)


def kernel(x, edges, W, dense_kernel, dense_bias):
    h2 = _tc_proj(x.astype(jnp.float32), W)
    parts, cnts = _sc_scatter(h2, jnp.ravel(edges))
    out = _tc_finish(parts, cnts, dense_kernel, dense_bias)
    return jnp.reshape(out, ())
